# baseline (speedup 1.0000x reference)
import functools

import jax
import jax.numpy as jnp
from jax import lax
from jax.experimental import pallas as pl
from jax.experimental.pallas import tpu as pltpu
from jax.experimental.pallas import tpu_sc as plsc

N = 10000
E = 320000
IN_DIM = 128
HID = 64
NC = 2
NS = 16
NW = NC * NS
L = 16
EPW = E // NW
CH = 80
NCH = EPW // CH
STRIPE = 624
TAIL0 = NS * STRIPE
TAILN = N - TAIL0
BN = 400
GRID = N // BN

_mesh = plsc.VectorSubcoreMesh(
    core_axis_name="c", subcore_axis_name="s", num_cores=NC, num_subcores=NS)
_sc_params = pltpu.CompilerParams(
    needs_layout_passes=False, use_tc_tiling_on_sc=False)


def _iota16():
    return lax.iota(jnp.int32, L)


def _splat(val):
    return jnp.full((L,), val, jnp.int32)


def _stripe_copy(src, dst, s):
    row0 = s * STRIPE
    pltpu.sync_copy(src.at[pl.ds(row0, STRIPE)], dst.at[pl.ds(row0, STRIPE)])

    @pl.when(s == 0)
    def _():
        pltpu.sync_copy(src.at[pl.ds(TAIL0, TAILN)], dst.at[pl.ds(TAIL0, TAILN)])


def _zero_cols(ref):
    @plsc.parallel_loop(0, CH, unroll=4)
    def _(r):
        plsc.store_scatter(ref, [_splat(0) + r, _iota16()],
                           jnp.zeros((L,), jnp.float32))


EXT = 80

@functools.partial(
    pl.kernel,
    out_type=[
        jax.ShapeDtypeStruct((NC, N, HID), jnp.float32),
        jax.ShapeDtypeStruct((NC, N, HID), jnp.float32),
        jax.ShapeDtypeStruct((NC, N, L), jnp.float32),
    ],
    mesh=_mesh,
    compiler_params=_sc_params,
    scratch_types=[
        pltpu.VMEM((2, 2, CH), jnp.int32),
        pltpu.VMEM((2, CH, EXT), jnp.float32),
        pltpu.VMEM((2, CH, EXT), jnp.float32),
        pltpu.VMEM((CH, HID), jnp.float32),
        pltpu.VMEM((CH, HID), jnp.float32),
        pltpu.VMEM((CH, L), jnp.float32),
        pltpu.VMEM((CH, L), jnp.float32),
        pltpu.VMEM_SHARED((N, HID), jnp.float32),
        pltpu.VMEM_SHARED((N, HID), jnp.float32),
        pltpu.VMEM_SHARED((N, L), jnp.float32),
        pltpu.SemaphoreType.DMA,
        pltpu.SemaphoreType.DMA,
        pltpu.SemaphoreType.DMA,
        pltpu.SemaphoreType.DMA,
        pltpu.SemaphoreType.DMA,
        pltpu.SemaphoreType.DMA,
    ],
)
def _gat_fu_sc(ei_hbm, hfx_hbm, hux_hbm, zbig_hbm, zden_hbm,
               accf_out, accu_out, den_out,
               eiv, rowsf, rowsu, sf, su, exf, exu,
               accf_sp, accu_sp, den_sp, semf, semu,
               sem1, sem2, sem3, sem4):
    c = lax.axis_index("c")
    s = lax.axis_index("s")
    wid = s * NC + c
    _stripe_copy(zbig_hbm, accf_sp, s)
    _stripe_copy(zbig_hbm, accu_sp, s)
    _stripe_copy(zden_hbm, den_sp, s)
    plsc.subcore_barrier()

    base0 = wid * EPW
    _zero_cols(exf)
    _zero_cols(exu)

    def stage_and_start(b, k):
        base = base0 + k * CH
        pltpu.sync_copy(ei_hbm.at[:, pl.ds(base, CH)], eiv.at[b])
        pltpu.make_async_copy(hfx_hbm.at[eiv.at[b, 0]], rowsf.at[b], semf).start()
        pltpu.make_async_copy(hux_hbm.at[eiv.at[b, 1]], rowsu.at[b], semu).start()

    def wait_gathers(b):
        pltpu.make_async_copy(hfx_hbm.at[eiv.at[b, 0]], rowsf.at[b], semf).wait()
        pltpu.make_async_copy(hux_hbm.at[eiv.at[b, 1]], rowsu.at[b], semu).wait()

    def compute_scatter(b):
        rf = rowsf.at[b]
        ru = rowsu.at[b]
        ef = exf
        eu = exu
        for g in range(CH // L):
            eidx = g * L + _iota16()
            for h in range(2):
                a = (plsc.load_gather(rf, [eidx, _splat(HID + h)])
                     + plsc.load_gather(ru, [eidx, _splat(HID + 2 + h)]))
                a = jnp.where(a >= 0.0, a, 0.2 * a)
                plsc.store_scatter(ef, [eidx, _splat(h)], jnp.exp(a))
                a = (plsc.load_gather(ru, [eidx, _splat(HID + h)])
                     + plsc.load_gather(rf, [eidx, _splat(HID + 2 + h)]))
                a = jnp.where(a >= 0.0, a, 0.2 * a)
                plsc.store_scatter(eu, [eidx, _splat(2 + h)], jnp.exp(a))

        sfb = sf
        sub = su

        @plsc.parallel_loop(0, CH // L, unroll=1)
        def _(g2):
            r16 = g2 * L + _iota16()
            f0 = plsc.load_gather(ef, [r16, _splat(0)])
            f1 = plsc.load_gather(ef, [r16, _splat(1)])
            u0 = plsc.load_gather(eu, [r16, _splat(2)])
            u1 = plsc.load_gather(eu, [r16, _splat(3)])
            for col in range(HID):
                cv = _splat(col)
                vf = plsc.load_gather(rf, [r16, cv]) * (f0 if col < 32 else f1)
                plsc.store_scatter(sfb, [r16, cv], vf)
                vu = plsc.load_gather(ru, [r16, cv]) * (u0 if col < 32 else u1)
                plsc.store_scatter(sub, [r16, cv], vu)

        d1 = pltpu.make_async_copy(sfb, accf_sp.at[eiv.at[b, 1]], sem1)
        d2 = pltpu.make_async_copy(ef, den_sp.at[eiv.at[b, 1]], sem2)
        d3 = pltpu.make_async_copy(sub, accu_sp.at[eiv.at[b, 0]], sem3)
        d4 = pltpu.make_async_copy(eu, den_sp.at[eiv.at[b, 0]], sem4)
        d1.start(add=True)
        d2.start(add=True)
        d3.start(add=True)
        d4.start(add=True)
        d1.wait()
        d2.wait()
        d3.wait()
        d4.wait()

    stage_and_start(0, 0)

    def pair_body(i, carry):
        k = 2 * i
        wait_gathers(0)
        stage_and_start(1, k + 1)
        compute_scatter(0)
        wait_gathers(1)
        stage_and_start(0, k + 2)
        compute_scatter(1)
        return carry

    lax.fori_loop(0, (NCH - 1) // 2, pair_body, 0)
    wait_gathers(0)
    compute_scatter(0)

    plsc.subcore_barrier()
    _stripe_copy(accf_sp, accf_out.at[c], s)
    _stripe_copy(accu_sp, accu_out.at[c], s)
    _stripe_copy(den_sp, den_out.at[c], s)


@functools.partial(
    pl.kernel,
    out_type=[jax.ShapeDtypeStruct((NC, N, L), jnp.float32)],
    mesh=_mesh,
    compiler_params=_sc_params,
    scratch_types=[
        pltpu.VMEM((N,), jnp.float32),
        pltpu.VMEM((L,), jnp.float32),
        pltpu.VMEM((2, 2, CH), jnp.int32),
        pltpu.VMEM((2, CH, L), jnp.float32),
        pltpu.VMEM_SHARED((N, L), jnp.float32),
    ],
)
def _gat_out_sc(ei_hbm, g_hbm, p_hbm, zden_hbm, nd_out,
                g_v, p_v, eiv, exbuf, nd_sp):
    c = lax.axis_index("c")
    s = lax.axis_index("s")
    wid = s * NC + c
    _stripe_copy(zden_hbm, nd_sp, s)
    pltpu.sync_copy(g_hbm, g_v)
    pltpu.sync_copy(p_hbm, p_v)
    plsc.subcore_barrier()

    base0 = wid * EPW
    _zero_cols(exbuf.at[0])
    _zero_cols(exbuf.at[1])

    def stage(b, k):
        base = base0 + k * CH
        pltpu.sync_copy(ei_hbm.at[:, pl.ds(base, CH)], eiv.at[b])

    def compute_scatter(b):
        eb = exbuf.at[b]
        aos = plsc.load_gather(p_v, [_splat(0)])
        aod = plsc.load_gather(p_v, [_splat(1)])

        @plsc.parallel_loop(0, CH // L, unroll=2)
        def _(g):
            sl = pl.ds(g * L, L)
            s16 = eiv[b, 0, sl]
            d16 = eiv[b, 1, sl]
            eidx = g * L + _iota16()
            gs = plsc.load_gather(g_v, [s16])
            gd = plsc.load_gather(g_v, [d16])
            a = aos * gs + aod * gd
            a = jnp.where(a >= 0.0, a, 0.2 * a)
            ex = jnp.exp(a)
            plsc.store_scatter(eb, [eidx, _splat(0)], ex * gs)
            plsc.store_scatter(eb, [eidx, _splat(1)], ex)

        pltpu.sync_copy(eb, nd_sp.at[eiv.at[b, 1]], add=True)

    stage(0, 0)

    def pair_body(i, carry):
        k = 2 * i
        stage(1, k + 1)
        compute_scatter(0)
        stage(0, k + 2)
        compute_scatter(1)
        return carry

    lax.fori_loop(0, (NCH - 1) // 2, pair_body, 0)
    compute_scatter(0)

    plsc.subcore_barrier()
    _stripe_copy(nd_sp, nd_out.at[c], s)


def _proj_body(x_ref, w2_ref, am_ref, hh_ref, al_ref, hfx_ref, hux_ref):
    hh = jnp.dot(x_ref[...], w2_ref[...], preferred_element_type=jnp.float32)
    hh_ref[...] = hh
    al = jnp.dot(hh, am_ref[...], preferred_element_type=jnp.float32)
    al_ref[...] = al
    pad = jnp.zeros((hh.shape[0], 12), jnp.float32)
    hfx_ref[...] = jnp.concatenate(
        [hh[:, :HID], al[:, 0:2], al[:, 6:8], pad], axis=1)
    hux_ref[...] = jnp.concatenate(
        [hh[:, HID:], al[:, 4:6], al[:, 2:4], pad], axis=1)


def _mix_body(hh_ref, al_ref, accf_ref, denf_ref, accu_ref, denu_ref,
              wfc_ref, bvec_ref, wo_ref, g_ref):
    hh = hh_ref[...]
    al = al_ref[...]
    bvec = bvec_ref[...]

    def layer(acc_ref, den_ref, a_self, hcols, boff, dcol=0):
        acc = acc_ref[0] + acc_ref[1]
        ex = jnp.exp(jnp.where(a_self >= 0.0, a_self, 0.2 * a_self))
        den = (den_ref[0][:, dcol:dcol + 2]
               + den_ref[1][:, dcol:dcol + 2] + ex)
        outs = []
        for h in range(2):
            hf_h = hcols[:, h * 32:(h + 1) * 32]
            num_h = acc[:, h * 32:(h + 1) * 32] + ex[:, h:h + 1] * hf_h
            outs.append(num_h / (den[:, h:h + 1] + 1e-16))
        out = jnp.concatenate(outs, axis=1) + bvec[:, boff:boff + HID]
        return jnp.maximum(out, 0.0)

    hF = layer(accf_ref, denf_ref, al[:, 0:2] + al[:, 2:4], hh[:, :HID], 0)
    hU = layer(accu_ref, denu_ref, al[:, 4:6] + al[:, 6:8], hh[:, HID:], HID,
               dcol=2)
    hcat = jnp.concatenate([hF, hU], axis=1)
    hmid = jnp.dot(hcat, wfc_ref[...], preferred_element_type=jnp.float32)
    hmid = jnp.maximum(hmid + bvec[:, 2 * HID:3 * HID], 0.0)
    g_ref[...] = jnp.dot(hmid, wo_ref[...], preferred_element_type=jnp.float32)


def _final_body(nd_ref, g_ref, sc_ref, out_ref):
    g = g_ref[...]
    nd = nd_ref[0] + nd_ref[1]
    aos = sc_ref[0, 0]
    aod = sc_ref[0, 1]
    bo = sc_ref[0, 2]
    a_self = (aos + aod) * g
    ex = jnp.exp(jnp.where(a_self >= 0.0, a_self, 0.2 * a_self))
    val = (nd[:, 0:1] + ex * g) / (nd[:, 1:2] + ex + 1e-16) + bo
    out_ref[...] = jax.nn.sigmoid(val)


def kernel(x, edge_index, Wf, af_src, af_dst, bf, Wu, au_src, au_dst, bu,
           Wfc, bfc, Wo, ao_src, ao_dst, bo):
    f32 = jnp.float32

    W2 = jnp.concatenate([Wf, Wu], axis=1)
    A = jnp.zeros((2 * HID, 8), f32)
    A = A.at[0:32, 0].set(af_src[0]).at[32:64, 1].set(af_src[1])
    A = A.at[0:32, 2].set(af_dst[0]).at[32:64, 3].set(af_dst[1])
    A = A.at[64:96, 4].set(au_src[0]).at[96:128, 5].set(au_src[1])
    A = A.at[64:96, 6].set(au_dst[0]).at[96:128, 7].set(au_dst[1])

    hh, al, hfx, hux = pl.pallas_call(
        _proj_body,
        grid=(GRID,),
        in_specs=[
            pl.BlockSpec((BN, IN_DIM), lambda i: (i, 0)),
            pl.BlockSpec((IN_DIM, 2 * HID), lambda i: (0, 0)),
            pl.BlockSpec((2 * HID, 8), lambda i: (0, 0)),
        ],
        out_specs=[
            pl.BlockSpec((BN, 2 * HID), lambda i: (i, 0)),
            pl.BlockSpec((BN, 8), lambda i: (i, 0)),
            pl.BlockSpec((BN, 80), lambda i: (i, 0)),
            pl.BlockSpec((BN, 80), lambda i: (i, 0)),
        ],
        out_shape=[
            jax.ShapeDtypeStruct((N, 2 * HID), f32),
            jax.ShapeDtypeStruct((N, 8), f32),
            jax.ShapeDtypeStruct((N, 80), f32),
            jax.ShapeDtypeStruct((N, 80), f32),
        ],
    )(x, W2, A)

    zbig = jnp.zeros((N, HID), f32)
    zden = jnp.zeros((N, L), f32)

    accF, accU, den2 = _gat_fu_sc(edge_index, hfx, hux, zbig, zden)

    bvec = jnp.concatenate([bf, bu, bfc]).reshape(1, 3 * HID)
    g = pl.pallas_call(
        _mix_body,
        grid=(GRID,),
        in_specs=[
            pl.BlockSpec((BN, 2 * HID), lambda i: (i, 0)),
            pl.BlockSpec((BN, 8), lambda i: (i, 0)),
            pl.BlockSpec((NC, BN, HID), lambda i: (0, i, 0)),
            pl.BlockSpec((NC, BN, L), lambda i: (0, i, 0)),
            pl.BlockSpec((NC, BN, HID), lambda i: (0, i, 0)),
            pl.BlockSpec((NC, BN, L), lambda i: (0, i, 0)),
            pl.BlockSpec((2 * HID, HID), lambda i: (0, 0)),
            pl.BlockSpec((1, 3 * HID), lambda i: (0, 0)),
            pl.BlockSpec((HID, 1), lambda i: (0, 0)),
        ],
        out_specs=pl.BlockSpec((BN, 1), lambda i: (i, 0)),
        out_shape=jax.ShapeDtypeStruct((N, 1), f32),
    )(hh, al, accF, den2, accU, den2, Wfc, bvec, Wo)

    gflat = g[:, 0]
    params = jnp.zeros((L,), f32).at[0].set(ao_src[0, 0]).at[1].set(ao_dst[0, 0])
    (nd,) = _gat_out_sc(edge_index, gflat, params, zden)

    scal = jnp.stack([ao_src[0, 0], ao_dst[0, 0], bo[0]]).reshape(1, 3)
    out = pl.pallas_call(
        _final_body,
        grid=(GRID,),
        in_specs=[
            pl.BlockSpec((NC, BN, L), lambda i: (0, i, 0)),
            pl.BlockSpec((BN, 1), lambda i: (i, 0)),
            pl.BlockSpec((1, 3), lambda i: (0, 0)),
        ],
        out_specs=pl.BlockSpec((BN, 1), lambda i: (i, 0)),
        out_shape=jax.ShapeDtypeStruct((N, 1), f32),
    )(nd, g, scal)
    return out


# speedup vs baseline: 2.5473x; 2.5473x over previous
import functools

import jax
import jax.numpy as jnp
from jax import lax
from jax.experimental import pallas as pl
from jax.experimental.pallas import tpu as pltpu
from jax.experimental.pallas import tpu_sc as plsc

N = 10000
E = 320000
IN_DIM = 128
HID = 64
NC = 2
NS = 16
NW = NC * NS
L = 16
EPW = E // NW
CH = 80
NCH = EPW // CH
STRIPE = 624
TAIL0 = NS * STRIPE
TAILN = N - TAIL0
BN = 400
GRID = N // BN

_mesh = plsc.VectorSubcoreMesh(
    core_axis_name="c", subcore_axis_name="s", num_cores=NC, num_subcores=NS)
_sc_params = pltpu.CompilerParams(
    needs_layout_passes=False, use_tc_tiling_on_sc=False)


def _iota16():
    return lax.iota(jnp.int32, L)


def _splat(val):
    return jnp.full((L,), val, jnp.int32)


def _stripe_copy(src, dst, s):
    row0 = s * STRIPE
    pltpu.sync_copy(src.at[pl.ds(row0, STRIPE)], dst.at[pl.ds(row0, STRIPE)])

    @pl.when(s == 0)
    def _():
        pltpu.sync_copy(src.at[pl.ds(TAIL0, TAILN)], dst.at[pl.ds(TAIL0, TAILN)])


def _zero_cols(ref):
    @plsc.parallel_loop(0, CH, unroll=4)
    def _(r):
        plsc.store_scatter(ref, [_splat(0) + r, _iota16()],
                           jnp.zeros((L,), jnp.float32))


EXT = 80

@functools.partial(
    pl.kernel,
    out_type=[
        jax.ShapeDtypeStruct((NC, N, HID), jnp.float32),
        jax.ShapeDtypeStruct((NC, N, HID), jnp.float32),
        jax.ShapeDtypeStruct((NC, N, L), jnp.float32),
    ],
    mesh=_mesh,
    compiler_params=_sc_params,
    scratch_types=[
        pltpu.VMEM((4, 2, CH), jnp.int32),
        pltpu.VMEM((2, CH, EXT), jnp.float32),
        pltpu.VMEM((2, CH, EXT), jnp.float32),
        pltpu.VMEM((CH, HID), jnp.float32),
        pltpu.VMEM((CH, HID), jnp.float32),
        pltpu.VMEM((CH, L), jnp.float32),
        pltpu.VMEM((CH, L), jnp.float32),
        pltpu.VMEM_SHARED((N, HID), jnp.float32),
        pltpu.VMEM_SHARED((N, HID), jnp.float32),
        pltpu.VMEM_SHARED((N, L), jnp.float32),
        pltpu.SemaphoreType.DMA,
        pltpu.SemaphoreType.DMA,
        pltpu.SemaphoreType.DMA,
        pltpu.SemaphoreType.DMA,
        pltpu.SemaphoreType.DMA,
        pltpu.SemaphoreType.DMA,
        pltpu.SemaphoreType.DMA,
    ],
)
def _gat_fu_sc(ei_hbm, hfx_hbm, hux_hbm, zbig_hbm, zden_hbm,
               accf_out, accu_out, den_out,
               eiv, rowsf, rowsu, sf, su, exf, exu,
               accf_sp, accu_sp, den_sp, semf, semu,
               sem1, sem2, sem3, sem4, semi):
    c = lax.axis_index("c")
    s = lax.axis_index("s")
    wid = s * NC + c
    _stripe_copy(zbig_hbm, accf_sp, s)
    _stripe_copy(zbig_hbm, accu_sp, s)
    _stripe_copy(zden_hbm, den_sp, s)
    plsc.subcore_barrier()

    base0 = wid * EPW
    _zero_cols(exf)
    _zero_cols(exu)

    def idx_desc(b4, k):
        base = base0 + k * CH
        return pltpu.make_async_copy(
            ei_hbm.at[:, pl.ds(base, CH)], eiv.at[b4], semi)

    def start_gathers(b4, rb):
        pltpu.make_async_copy(
            hfx_hbm.at[eiv.at[b4, 0]], rowsf.at[rb], semf).start()
        pltpu.make_async_copy(
            hux_hbm.at[eiv.at[b4, 1]], rowsu.at[rb], semu).start()

    def wait_gathers(b4, rb):
        pltpu.make_async_copy(
            hfx_hbm.at[eiv.at[b4, 0]], rowsf.at[rb], semf).wait()
        pltpu.make_async_copy(
            hux_hbm.at[eiv.at[b4, 1]], rowsu.at[rb], semu).wait()

    def compute_scatter(b4, rb):
        b = b4
        rf = rowsf.at[rb]
        ru = rowsu.at[rb]
        ef = exf
        eu = exu
        for g in range(CH // L):
            eidx = g * L + _iota16()
            for h in range(2):
                a = (plsc.load_gather(rf, [eidx, _splat(HID + h)])
                     + plsc.load_gather(ru, [eidx, _splat(HID + 2 + h)]))
                a = jnp.where(a >= 0.0, a, 0.2 * a)
                plsc.store_scatter(ef, [eidx, _splat(h)], jnp.exp(a))
                a = (plsc.load_gather(ru, [eidx, _splat(HID + h)])
                     + plsc.load_gather(rf, [eidx, _splat(HID + 2 + h)]))
                a = jnp.where(a >= 0.0, a, 0.2 * a)
                plsc.store_scatter(eu, [eidx, _splat(2 + h)], jnp.exp(a))

        sfb = sf
        sub = su

        @plsc.parallel_loop(0, CH, unroll=2)
        def _(r):
            rsp = _splat(0) + r
            f0 = plsc.load_gather(ef, [rsp, _splat(0)])
            f1 = plsc.load_gather(ef, [rsp, _splat(1)])
            u0 = plsc.load_gather(eu, [rsp, _splat(2)])
            u1 = plsc.load_gather(eu, [rsp, _splat(3)])
            for q in range(HID // L):
                colv = q * L + _iota16()
                vf = plsc.load_gather(rf, [rsp, colv]) * (f0 if q < 2 else f1)
                plsc.store_scatter(sfb, [rsp, colv], vf)
                vu = plsc.load_gather(ru, [rsp, colv]) * (u0 if q < 2 else u1)
                plsc.store_scatter(sub, [rsp, colv], vu)

        d1 = pltpu.make_async_copy(sfb, accf_sp.at[eiv.at[b, 1]], sem1)
        d2 = pltpu.make_async_copy(ef, den_sp.at[eiv.at[b, 1]], sem2)
        d3 = pltpu.make_async_copy(sub, accu_sp.at[eiv.at[b, 0]], sem3)
        d4 = pltpu.make_async_copy(eu, den_sp.at[eiv.at[b, 0]], sem4)
        d1.start(add=True)
        d2.start(add=True)
        d3.start(add=True)
        d4.start(add=True)
        d1.wait()
        d2.wait()
        d3.wait()
        d4.wait()

    d0 = idx_desc(0, 0)
    d0.start()
    d0.wait()
    start_gathers(0, 0)
    idx_desc(1, 1).start()

    def process(k, b4, i):
        rb = (k % 2)
        wait_gathers(b4, rb)
        idx_desc((b4 + 1) % 4, k + 1).wait()
        start_gathers((b4 + 1) % 4, (k + 1) % 2)

        @pl.when(i * 4 + b4 + 2 <= NCH - 1)
        def _():
            idx_desc((b4 + 2) % 4, k + 2).start()

        compute_scatter(b4, rb)

    def quad_body(i, carry):
        k = 4 * i
        for j in range(4):
            process(k + j, j, i)
        return carry

    lax.fori_loop(0, (NCH - 1) // 4, quad_body, 0)
    wait_gathers(0, 0)
    compute_scatter(0, 0)

    plsc.subcore_barrier()
    _stripe_copy(accf_sp, accf_out.at[c], s)
    _stripe_copy(accu_sp, accu_out.at[c], s)
    _stripe_copy(den_sp, den_out.at[c], s)


@functools.partial(
    pl.kernel,
    out_type=[jax.ShapeDtypeStruct((NC, N, L), jnp.float32)],
    mesh=_mesh,
    compiler_params=_sc_params,
    scratch_types=[
        pltpu.VMEM((N,), jnp.float32),
        pltpu.VMEM((L,), jnp.float32),
        pltpu.VMEM((2, 2, CH), jnp.int32),
        pltpu.VMEM((2, CH, L), jnp.float32),
        pltpu.VMEM_SHARED((N, L), jnp.float32),
    ],
)
def _gat_out_sc(ei_hbm, g_hbm, p_hbm, zden_hbm, nd_out,
                g_v, p_v, eiv, exbuf, nd_sp):
    c = lax.axis_index("c")
    s = lax.axis_index("s")
    wid = s * NC + c
    _stripe_copy(zden_hbm, nd_sp, s)
    pltpu.sync_copy(g_hbm, g_v)
    pltpu.sync_copy(p_hbm, p_v)
    plsc.subcore_barrier()

    base0 = wid * EPW
    _zero_cols(exbuf.at[0])
    _zero_cols(exbuf.at[1])

    def stage(b, k):
        base = base0 + k * CH
        pltpu.sync_copy(ei_hbm.at[:, pl.ds(base, CH)], eiv.at[b])

    def compute_scatter(b):
        eb = exbuf.at[b]
        aos = plsc.load_gather(p_v, [_splat(0)])
        aod = plsc.load_gather(p_v, [_splat(1)])

        @plsc.parallel_loop(0, CH // L, unroll=2)
        def _(g):
            sl = pl.ds(g * L, L)
            s16 = eiv[b, 0, sl]
            d16 = eiv[b, 1, sl]
            eidx = g * L + _iota16()
            gs = plsc.load_gather(g_v, [s16])
            gd = plsc.load_gather(g_v, [d16])
            a = aos * gs + aod * gd
            a = jnp.where(a >= 0.0, a, 0.2 * a)
            ex = jnp.exp(a)
            plsc.store_scatter(eb, [eidx, _splat(0)], ex * gs)
            plsc.store_scatter(eb, [eidx, _splat(1)], ex)

        pltpu.sync_copy(eb, nd_sp.at[eiv.at[b, 1]], add=True)

    stage(0, 0)

    def pair_body(i, carry):
        k = 2 * i
        stage(1, k + 1)
        compute_scatter(0)
        stage(0, k + 2)
        compute_scatter(1)
        return carry

    lax.fori_loop(0, (NCH - 1) // 2, pair_body, 0)
    compute_scatter(0)

    plsc.subcore_barrier()
    _stripe_copy(nd_sp, nd_out.at[c], s)


def _proj_body(x_ref, w2_ref, am_ref, hh_ref, al_ref, hfx_ref, hux_ref):
    hh = jnp.dot(x_ref[...], w2_ref[...], preferred_element_type=jnp.float32)
    hh_ref[...] = hh
    al = jnp.dot(hh, am_ref[...], preferred_element_type=jnp.float32)
    al_ref[...] = al
    pad = jnp.zeros((hh.shape[0], 12), jnp.float32)
    hfx_ref[...] = jnp.concatenate(
        [hh[:, :HID], al[:, 0:2], al[:, 6:8], pad], axis=1)
    hux_ref[...] = jnp.concatenate(
        [hh[:, HID:], al[:, 4:6], al[:, 2:4], pad], axis=1)


def _mix_body(hh_ref, al_ref, accf_ref, denf_ref, accu_ref, denu_ref,
              wfc_ref, bvec_ref, wo_ref, g_ref):
    hh = hh_ref[...]
    al = al_ref[...]
    bvec = bvec_ref[...]

    def layer(acc_ref, den_ref, a_self, hcols, boff, dcol=0):
        acc = acc_ref[0] + acc_ref[1]
        ex = jnp.exp(jnp.where(a_self >= 0.0, a_self, 0.2 * a_self))
        den = (den_ref[0][:, dcol:dcol + 2]
               + den_ref[1][:, dcol:dcol + 2] + ex)
        outs = []
        for h in range(2):
            hf_h = hcols[:, h * 32:(h + 1) * 32]
            num_h = acc[:, h * 32:(h + 1) * 32] + ex[:, h:h + 1] * hf_h
            outs.append(num_h / (den[:, h:h + 1] + 1e-16))
        out = jnp.concatenate(outs, axis=1) + bvec[:, boff:boff + HID]
        return jnp.maximum(out, 0.0)

    hF = layer(accf_ref, denf_ref, al[:, 0:2] + al[:, 2:4], hh[:, :HID], 0)
    hU = layer(accu_ref, denu_ref, al[:, 4:6] + al[:, 6:8], hh[:, HID:], HID,
               dcol=2)
    hcat = jnp.concatenate([hF, hU], axis=1)
    hmid = jnp.dot(hcat, wfc_ref[...], preferred_element_type=jnp.float32)
    hmid = jnp.maximum(hmid + bvec[:, 2 * HID:3 * HID], 0.0)
    g_ref[...] = jnp.dot(hmid, wo_ref[...], preferred_element_type=jnp.float32)


def _final_body(nd_ref, g_ref, sc_ref, out_ref):
    g = g_ref[...]
    nd = nd_ref[0] + nd_ref[1]
    aos = sc_ref[0, 0]
    aod = sc_ref[0, 1]
    bo = sc_ref[0, 2]
    a_self = (aos + aod) * g
    ex = jnp.exp(jnp.where(a_self >= 0.0, a_self, 0.2 * a_self))
    val = (nd[:, 0:1] + ex * g) / (nd[:, 1:2] + ex + 1e-16) + bo
    out_ref[...] = jax.nn.sigmoid(val)


def kernel(x, edge_index, Wf, af_src, af_dst, bf, Wu, au_src, au_dst, bu,
           Wfc, bfc, Wo, ao_src, ao_dst, bo):
    f32 = jnp.float32

    W2 = jnp.concatenate([Wf, Wu], axis=1)
    A = jnp.zeros((2 * HID, 8), f32)
    A = A.at[0:32, 0].set(af_src[0]).at[32:64, 1].set(af_src[1])
    A = A.at[0:32, 2].set(af_dst[0]).at[32:64, 3].set(af_dst[1])
    A = A.at[64:96, 4].set(au_src[0]).at[96:128, 5].set(au_src[1])
    A = A.at[64:96, 6].set(au_dst[0]).at[96:128, 7].set(au_dst[1])

    hh, al, hfx, hux = pl.pallas_call(
        _proj_body,
        grid=(GRID,),
        in_specs=[
            pl.BlockSpec((BN, IN_DIM), lambda i: (i, 0)),
            pl.BlockSpec((IN_DIM, 2 * HID), lambda i: (0, 0)),
            pl.BlockSpec((2 * HID, 8), lambda i: (0, 0)),
        ],
        out_specs=[
            pl.BlockSpec((BN, 2 * HID), lambda i: (i, 0)),
            pl.BlockSpec((BN, 8), lambda i: (i, 0)),
            pl.BlockSpec((BN, 80), lambda i: (i, 0)),
            pl.BlockSpec((BN, 80), lambda i: (i, 0)),
        ],
        out_shape=[
            jax.ShapeDtypeStruct((N, 2 * HID), f32),
            jax.ShapeDtypeStruct((N, 8), f32),
            jax.ShapeDtypeStruct((N, 80), f32),
            jax.ShapeDtypeStruct((N, 80), f32),
        ],
    )(x, W2, A)

    zbig = jnp.zeros((N, HID), f32)
    zden = jnp.zeros((N, L), f32)

    accF, accU, den2 = _gat_fu_sc(edge_index, hfx, hux, zbig, zden)

    bvec = jnp.concatenate([bf, bu, bfc]).reshape(1, 3 * HID)
    g = pl.pallas_call(
        _mix_body,
        grid=(GRID,),
        in_specs=[
            pl.BlockSpec((BN, 2 * HID), lambda i: (i, 0)),
            pl.BlockSpec((BN, 8), lambda i: (i, 0)),
            pl.BlockSpec((NC, BN, HID), lambda i: (0, i, 0)),
            pl.BlockSpec((NC, BN, L), lambda i: (0, i, 0)),
            pl.BlockSpec((NC, BN, HID), lambda i: (0, i, 0)),
            pl.BlockSpec((NC, BN, L), lambda i: (0, i, 0)),
            pl.BlockSpec((2 * HID, HID), lambda i: (0, 0)),
            pl.BlockSpec((1, 3 * HID), lambda i: (0, 0)),
            pl.BlockSpec((HID, 1), lambda i: (0, 0)),
        ],
        out_specs=pl.BlockSpec((BN, 1), lambda i: (i, 0)),
        out_shape=jax.ShapeDtypeStruct((N, 1), f32),
    )(hh, al, accF, den2, accU, den2, Wfc, bvec, Wo)

    gflat = g[:, 0]
    params = jnp.zeros((L,), f32).at[0].set(ao_src[0, 0]).at[1].set(ao_dst[0, 0])
    (nd,) = _gat_out_sc(edge_index, gflat, params, zden)

    scal = jnp.stack([ao_src[0, 0], ao_dst[0, 0], bo[0]]).reshape(1, 3)
    out = pl.pallas_call(
        _final_body,
        grid=(GRID,),
        in_specs=[
            pl.BlockSpec((NC, BN, L), lambda i: (0, i, 0)),
            pl.BlockSpec((BN, 1), lambda i: (i, 0)),
            pl.BlockSpec((1, 3), lambda i: (0, 0)),
        ],
        out_specs=pl.BlockSpec((BN, 1), lambda i: (i, 0)),
        out_shape=jax.ShapeDtypeStruct((N, 1), f32),
    )(nd, g, scal)
    return out


# speedup vs baseline: 2.6790x; 1.0517x over previous
import functools

import jax
import jax.numpy as jnp
from jax import lax
from jax.experimental import pallas as pl
from jax.experimental.pallas import tpu as pltpu
from jax.experimental.pallas import tpu_sc as plsc

N = 10000
E = 320000
IN_DIM = 128
HID = 64
NC = 2
NS = 16
NW = NC * NS
L = 16
EPW = E // NW
CH = 80
NCH = EPW // CH
STRIPE = 624
TAIL0 = NS * STRIPE
TAILN = N - TAIL0
BN = 400
GRID = N // BN

_mesh = plsc.VectorSubcoreMesh(
    core_axis_name="c", subcore_axis_name="s", num_cores=NC, num_subcores=NS)
_sc_params = pltpu.CompilerParams(
    needs_layout_passes=False, use_tc_tiling_on_sc=False)


def _iota16():
    return lax.iota(jnp.int32, L)


def _splat(val):
    return jnp.full((L,), val, jnp.int32)


def _stripe_copy(src, dst, s):
    row0 = s * STRIPE
    pltpu.sync_copy(src.at[pl.ds(row0, STRIPE)], dst.at[pl.ds(row0, STRIPE)])

    @pl.when(s == 0)
    def _():
        pltpu.sync_copy(src.at[pl.ds(TAIL0, TAILN)], dst.at[pl.ds(TAIL0, TAILN)])


def _zero_cols(ref):
    @plsc.parallel_loop(0, CH, unroll=4)
    def _(r):
        plsc.store_scatter(ref, [_splat(0) + r, _iota16()],
                           jnp.zeros((L,), jnp.float32))


EXT = 80

@functools.partial(
    pl.kernel,
    out_type=[
        jax.ShapeDtypeStruct((NC, N, HID), jnp.float32),
        jax.ShapeDtypeStruct((NC, N, HID), jnp.float32),
        jax.ShapeDtypeStruct((NC, N, L), jnp.float32),
    ],
    mesh=_mesh,
    compiler_params=_sc_params,
    scratch_types=[
        pltpu.VMEM((4, 2, CH), jnp.int32),
        pltpu.VMEM((2, CH, EXT), jnp.float32),
        pltpu.VMEM((2, CH, EXT), jnp.float32),
        pltpu.VMEM((CH, HID), jnp.float32),
        pltpu.VMEM((CH, HID), jnp.float32),
        pltpu.VMEM((CH, L), jnp.float32),
        pltpu.VMEM((CH, L), jnp.float32),
        pltpu.VMEM_SHARED((N, HID), jnp.float32),
        pltpu.VMEM_SHARED((N, HID), jnp.float32),
        pltpu.VMEM_SHARED((N, L), jnp.float32),
        pltpu.SemaphoreType.DMA,
        pltpu.SemaphoreType.DMA,
        pltpu.SemaphoreType.DMA,
        pltpu.SemaphoreType.DMA,
        pltpu.SemaphoreType.DMA,
        pltpu.SemaphoreType.DMA,
        pltpu.SemaphoreType.DMA,
    ],
)
def _gat_fu_sc(ei_hbm, hfx_hbm, hux_hbm, zbig_hbm, zden_hbm,
               accf_out, accu_out, den_out,
               eiv, rowsf, rowsu, sf, su, exf, exu,
               accf_sp, accu_sp, den_sp, semf, semu,
               sem1, sem2, sem3, sem4, semi):
    c = lax.axis_index("c")
    s = lax.axis_index("s")
    wid = s * NC + c
    _stripe_copy(zbig_hbm, accf_sp, s)
    _stripe_copy(zbig_hbm, accu_sp, s)
    _stripe_copy(zden_hbm, den_sp, s)
    plsc.subcore_barrier()

    base0 = wid * EPW
    _zero_cols(exf)
    _zero_cols(exu)

    def idx_desc(b4, k):
        base = base0 + k * CH
        return pltpu.make_async_copy(
            ei_hbm.at[:, pl.ds(base, CH)], eiv.at[b4], semi)

    def start_gathers(b4, rb):
        pltpu.make_async_copy(
            hfx_hbm.at[eiv.at[b4, 0]], rowsf.at[rb], semf).start()
        pltpu.make_async_copy(
            hux_hbm.at[eiv.at[b4, 1]], rowsu.at[rb], semu).start()

    def wait_gathers(b4, rb):
        pltpu.make_async_copy(
            hfx_hbm.at[eiv.at[b4, 0]], rowsf.at[rb], semf).wait()
        pltpu.make_async_copy(
            hux_hbm.at[eiv.at[b4, 1]], rowsu.at[rb], semu).wait()

    def compute_scatter(b4, rb):
        b = b4
        rf = rowsf.at[rb]
        ru = rowsu.at[rb]
        ef = exf
        eu = exu
        for g in range(CH // L):
            eidx = g * L + _iota16()
            for h in range(2):
                a = (plsc.load_gather(rf, [eidx, _splat(HID + h)])
                     + plsc.load_gather(ru, [eidx, _splat(HID + 2 + h)]))
                a = jnp.where(a >= 0.0, a, 0.2 * a)
                plsc.store_scatter(ef, [eidx, _splat(h)], jnp.exp(a))
                a = (plsc.load_gather(ru, [eidx, _splat(HID + h)])
                     + plsc.load_gather(rf, [eidx, _splat(HID + 2 + h)]))
                a = jnp.where(a >= 0.0, a, 0.2 * a)
                plsc.store_scatter(eu, [eidx, _splat(2 + h)], jnp.exp(a))

        sfb = sf
        sub = su

        @plsc.parallel_loop(0, CH, unroll=2)
        def _(r):
            rsp = _splat(0) + r
            f0 = plsc.load_gather(ef, [rsp, _splat(0)])
            f1 = plsc.load_gather(ef, [rsp, _splat(1)])
            u0 = plsc.load_gather(eu, [rsp, _splat(2)])
            u1 = plsc.load_gather(eu, [rsp, _splat(3)])
            for q in range(HID // L):
                colv = q * L + _iota16()
                vf = plsc.load_gather(rf, [rsp, colv]) * (f0 if q < 2 else f1)
                plsc.store_scatter(sfb, [rsp, colv], vf)
                vu = plsc.load_gather(ru, [rsp, colv]) * (u0 if q < 2 else u1)
                plsc.store_scatter(sub, [rsp, colv], vu)

        d1 = pltpu.make_async_copy(sfb, accf_sp.at[eiv.at[b, 1]], sem1)
        d2 = pltpu.make_async_copy(ef, den_sp.at[eiv.at[b, 1]], sem2)
        d3 = pltpu.make_async_copy(sub, accu_sp.at[eiv.at[b, 0]], sem3)
        d4 = pltpu.make_async_copy(eu, den_sp.at[eiv.at[b, 0]], sem4)
        d1.start(add=True)
        d2.start(add=True)
        d3.start(add=True)
        d4.start(add=True)
        d1.wait()
        d2.wait()
        d3.wait()
        d4.wait()

    d0 = idx_desc(0, 0)
    d0.start()
    d0.wait()
    start_gathers(0, 0)
    idx_desc(1, 1).start()

    def process(k, b4, i):
        rb = b4 % 2
        wait_gathers(b4, rb)
        idx_desc((b4 + 1) % 4, k + 1).wait()
        start_gathers((b4 + 1) % 4, (b4 + 1) % 2)

        @pl.when(i * 4 + b4 + 2 <= NCH - 1)
        def _():
            idx_desc((b4 + 2) % 4, k + 2).start()

        compute_scatter(b4, rb)

    def quad_body(i, carry):
        k = 4 * i
        for j in range(4):
            process(k + j, j, i)
        return carry

    lax.fori_loop(0, (NCH - 1) // 4, quad_body, 0)
    wait_gathers(0, 0)
    compute_scatter(0, 0)

    plsc.subcore_barrier()
    _stripe_copy(accf_sp, accf_out.at[c], s)
    _stripe_copy(accu_sp, accu_out.at[c], s)
    _stripe_copy(den_sp, den_out.at[c], s)


@functools.partial(
    pl.kernel,
    out_type=[jax.ShapeDtypeStruct((NC, N, L), jnp.float32)],
    mesh=_mesh,
    compiler_params=_sc_params,
    scratch_types=[
        pltpu.VMEM((N,), jnp.float32),
        pltpu.VMEM((L,), jnp.float32),
        pltpu.VMEM((4, 2, CH), jnp.int32),
        pltpu.VMEM((2, CH, L), jnp.float32),
        pltpu.VMEM_SHARED((N, L), jnp.float32),
        pltpu.SemaphoreType.DMA,
        pltpu.SemaphoreType.DMA,
        pltpu.SemaphoreType.DMA,
    ],
)
def _gat_out_sc(ei_hbm, g_hbm, p_hbm, zden_hbm, nd_out,
                g_v, p_v, eiv, exbuf, nd_sp, semi, sems0, sems1):
    c = lax.axis_index("c")
    s = lax.axis_index("s")
    wid = s * NC + c
    _stripe_copy(zden_hbm, nd_sp, s)
    pltpu.sync_copy(g_hbm, g_v)
    pltpu.sync_copy(p_hbm, p_v)
    plsc.subcore_barrier()

    base0 = wid * EPW
    _zero_cols(exbuf.at[0])
    _zero_cols(exbuf.at[1])
    sems = (sems0, sems1)

    def idx_desc(b4, k):
        base = base0 + k * CH
        return pltpu.make_async_copy(
            ei_hbm.at[:, pl.ds(base, CH)], eiv.at[b4], semi)

    def scat_desc(b4, eb):
        return pltpu.make_async_copy(
            exbuf.at[eb], nd_sp.at[eiv.at[b4, 1]], sems[eb])

    def compute(b4, eb):
        ebr = exbuf.at[eb]
        aos = plsc.load_gather(p_v, [_splat(0)])
        aod = plsc.load_gather(p_v, [_splat(1)])

        @plsc.parallel_loop(0, CH // L, unroll=2)
        def _(g):
            sl = pl.ds(g * L, L)
            s16 = eiv[b4, 0, sl]
            d16 = eiv[b4, 1, sl]
            eidx = g * L + _iota16()
            gs = plsc.load_gather(g_v, [s16])
            gd = plsc.load_gather(g_v, [d16])
            a = aos * gs + aod * gd
            a = jnp.where(a >= 0.0, a, 0.2 * a)
            ex = jnp.exp(a)
            plsc.store_scatter(ebr, [eidx, _splat(0)], ex * gs)
            plsc.store_scatter(ebr, [eidx, _splat(1)], ex)

    d0 = idx_desc(0, 0)
    d0.start()
    d0.wait()
    idx_desc(1, 1).start()

    def process(k, b4, i):
        eb = b4 % 2
        kk = i * 4 + b4

        @pl.when(kk >= 2)
        def _():
            scat_desc(b4, eb).wait()

        idx_desc((b4 + 1) % 4, k + 1).wait()

        @pl.when(kk + 2 <= NCH - 1)
        def _():
            idx_desc((b4 + 2) % 4, k + 2).start()

        compute(b4, eb)
        scat_desc(b4, eb).start(add=True)

    def quad_body(i, carry):
        k = 4 * i
        for j in range(4):
            process(k + j, j, i)
        return carry

    lax.fori_loop(0, (NCH - 1) // 4, quad_body, 0)
    scat_desc(0, 0).wait()
    compute(0, 0)
    scat_desc(0, 0).start(add=True)
    scat_desc(0, 1).wait()
    scat_desc(0, 0).wait()

    plsc.subcore_barrier()
    _stripe_copy(nd_sp, nd_out.at[c], s)


def _proj_body(x_ref, w2_ref, am_ref, hh_ref, al_ref, hfx_ref, hux_ref):
    hh = jnp.dot(x_ref[...], w2_ref[...], preferred_element_type=jnp.float32)
    hh_ref[...] = hh
    al = jnp.dot(hh, am_ref[...], preferred_element_type=jnp.float32)
    al_ref[...] = al
    pad = jnp.zeros((hh.shape[0], 12), jnp.float32)
    hfx_ref[...] = jnp.concatenate(
        [hh[:, :HID], al[:, 0:2], al[:, 6:8], pad], axis=1)
    hux_ref[...] = jnp.concatenate(
        [hh[:, HID:], al[:, 4:6], al[:, 2:4], pad], axis=1)


def _mix_body(hh_ref, al_ref, accf_ref, denf_ref, accu_ref, denu_ref,
              wfc_ref, bvec_ref, wo_ref, g_ref):
    hh = hh_ref[...]
    al = al_ref[...]
    bvec = bvec_ref[...]

    def layer(acc_ref, den_ref, a_self, hcols, boff, dcol=0):
        acc = acc_ref[0] + acc_ref[1]
        ex = jnp.exp(jnp.where(a_self >= 0.0, a_self, 0.2 * a_self))
        den = (den_ref[0][:, dcol:dcol + 2]
               + den_ref[1][:, dcol:dcol + 2] + ex)
        outs = []
        for h in range(2):
            hf_h = hcols[:, h * 32:(h + 1) * 32]
            num_h = acc[:, h * 32:(h + 1) * 32] + ex[:, h:h + 1] * hf_h
            outs.append(num_h / (den[:, h:h + 1] + 1e-16))
        out = jnp.concatenate(outs, axis=1) + bvec[:, boff:boff + HID]
        return jnp.maximum(out, 0.0)

    hF = layer(accf_ref, denf_ref, al[:, 0:2] + al[:, 2:4], hh[:, :HID], 0)
    hU = layer(accu_ref, denu_ref, al[:, 4:6] + al[:, 6:8], hh[:, HID:], HID,
               dcol=2)
    hcat = jnp.concatenate([hF, hU], axis=1)
    hmid = jnp.dot(hcat, wfc_ref[...], preferred_element_type=jnp.float32)
    hmid = jnp.maximum(hmid + bvec[:, 2 * HID:3 * HID], 0.0)
    g_ref[...] = jnp.dot(hmid, wo_ref[...], preferred_element_type=jnp.float32)


def _final_body(nd_ref, g_ref, sc_ref, out_ref):
    g = g_ref[...]
    nd = nd_ref[0] + nd_ref[1]
    aos = sc_ref[0, 0]
    aod = sc_ref[0, 1]
    bo = sc_ref[0, 2]
    a_self = (aos + aod) * g
    ex = jnp.exp(jnp.where(a_self >= 0.0, a_self, 0.2 * a_self))
    val = (nd[:, 0:1] + ex * g) / (nd[:, 1:2] + ex + 1e-16) + bo
    out_ref[...] = jax.nn.sigmoid(val)


def kernel(x, edge_index, Wf, af_src, af_dst, bf, Wu, au_src, au_dst, bu,
           Wfc, bfc, Wo, ao_src, ao_dst, bo):
    f32 = jnp.float32

    W2 = jnp.concatenate([Wf, Wu], axis=1)
    A = jnp.zeros((2 * HID, 8), f32)
    A = A.at[0:32, 0].set(af_src[0]).at[32:64, 1].set(af_src[1])
    A = A.at[0:32, 2].set(af_dst[0]).at[32:64, 3].set(af_dst[1])
    A = A.at[64:96, 4].set(au_src[0]).at[96:128, 5].set(au_src[1])
    A = A.at[64:96, 6].set(au_dst[0]).at[96:128, 7].set(au_dst[1])

    hh, al, hfx, hux = pl.pallas_call(
        _proj_body,
        grid=(GRID,),
        in_specs=[
            pl.BlockSpec((BN, IN_DIM), lambda i: (i, 0)),
            pl.BlockSpec((IN_DIM, 2 * HID), lambda i: (0, 0)),
            pl.BlockSpec((2 * HID, 8), lambda i: (0, 0)),
        ],
        out_specs=[
            pl.BlockSpec((BN, 2 * HID), lambda i: (i, 0)),
            pl.BlockSpec((BN, 8), lambda i: (i, 0)),
            pl.BlockSpec((BN, 80), lambda i: (i, 0)),
            pl.BlockSpec((BN, 80), lambda i: (i, 0)),
        ],
        out_shape=[
            jax.ShapeDtypeStruct((N, 2 * HID), f32),
            jax.ShapeDtypeStruct((N, 8), f32),
            jax.ShapeDtypeStruct((N, 80), f32),
            jax.ShapeDtypeStruct((N, 80), f32),
        ],
    )(x, W2, A)

    zbig = jnp.zeros((N, HID), f32)
    zden = jnp.zeros((N, L), f32)

    accF, accU, den2 = _gat_fu_sc(edge_index, hfx, hux, zbig, zden)

    bvec = jnp.concatenate([bf, bu, bfc]).reshape(1, 3 * HID)
    g = pl.pallas_call(
        _mix_body,
        grid=(GRID,),
        in_specs=[
            pl.BlockSpec((BN, 2 * HID), lambda i: (i, 0)),
            pl.BlockSpec((BN, 8), lambda i: (i, 0)),
            pl.BlockSpec((NC, BN, HID), lambda i: (0, i, 0)),
            pl.BlockSpec((NC, BN, L), lambda i: (0, i, 0)),
            pl.BlockSpec((NC, BN, HID), lambda i: (0, i, 0)),
            pl.BlockSpec((NC, BN, L), lambda i: (0, i, 0)),
            pl.BlockSpec((2 * HID, HID), lambda i: (0, 0)),
            pl.BlockSpec((1, 3 * HID), lambda i: (0, 0)),
            pl.BlockSpec((HID, 1), lambda i: (0, 0)),
        ],
        out_specs=pl.BlockSpec((BN, 1), lambda i: (i, 0)),
        out_shape=jax.ShapeDtypeStruct((N, 1), f32),
    )(hh, al, accF, den2, accU, den2, Wfc, bvec, Wo)

    gflat = g[:, 0]
    params = jnp.zeros((L,), f32).at[0].set(ao_src[0, 0]).at[1].set(ao_dst[0, 0])
    (nd,) = _gat_out_sc(edge_index, gflat, params, zden)

    scal = jnp.stack([ao_src[0, 0], ao_dst[0, 0], bo[0]]).reshape(1, 3)
    out = pl.pallas_call(
        _final_body,
        grid=(GRID,),
        in_specs=[
            pl.BlockSpec((NC, BN, L), lambda i: (0, i, 0)),
            pl.BlockSpec((BN, 1), lambda i: (i, 0)),
            pl.BlockSpec((1, 3), lambda i: (0, 0)),
        ],
        out_specs=pl.BlockSpec((BN, 1), lambda i: (i, 0)),
        out_shape=jax.ShapeDtypeStruct((N, 1), f32),
    )(nd, g, scal)
    return out
